# baseline (speedup 1.0000x reference)
import math

import jax
import jax.numpy as jnp
from jax import lax
from jax.experimental import pallas as pl
from jax.experimental.pallas import tpu as pltpu
from jax.experimental.pallas import tpu_sc as plsc

L, D = 2048, 768
DC = D // 2
H = 12
DH = DC // H
DFF = 1536
E = 8
BQ = 2048
BM = 256
NP = 2 * L + E * BM
NB = NP // BM
NPAIR = 2 * L
DI = D // 2
SC_CORES = 2
SC_SUBCORES = 16
NW = SC_CORES * SC_SUBCORES
PPW = NPAIR // NW
F32 = jnp.float32
BF16 = jnp.bfloat16


def _dot(a, b):
    return jnp.dot(a, b, preferred_element_type=F32)


def _bdot(a, b):
    return jnp.dot(a.astype(BF16), b.astype(BF16), preferred_element_type=F32)


def _ka(x1_ref, qw_ref, qb_ref, kw_ref, kb_ref, w0_ref, w1_ref, w2_ref,
        sb_ref, q_out, k_out, sal_out):
    x1 = x1_ref[...]
    q_out[...] = (_bdot(x1, qw_ref[...]) + qb_ref[...]) * (1.0 / math.sqrt(DH))
    k_out[...] = _bdot(x1, kw_ref[...]) + kb_ref[...]
    a0 = _bdot(x1, w0_ref[...])
    a1 = _bdot(x1, w1_ref[...])
    a2 = _bdot(x1, w2_ref[...])
    z = jnp.zeros((1, H), F32)
    sal_out[...] = (jnp.concatenate([z, a0[:-1]], axis=0) + a1
                    + jnp.concatenate([a2[1:], z], axis=0) + sb_ref[...])


def _kb(q_ref, k_ref, sal_ref, attw_ref, ctx_ref):
    q = q_ref[0]
    k = k_ref[0]
    s = _bdot(q, k.T) + sal_ref[0]
    m = jnp.max(s, axis=-1, keepdims=True)
    p = jnp.exp(s - m)
    attw = p * (1.0 / jnp.sum(p, axis=-1, keepdims=True))
    attw_ref[0] = attw
    ctx_ref[0] = _bdot(attw, k)


def _kc(ctx_ref, x1_ref, x2_ref, src_ref, stw_ref, stb_ref, w1_ref, w2_ref,
        ob_ref, l1g_ref, l1b_ref, gw_ref, gb_ref,
        h1_ref, mean_ref, cnt_ref, slot_ref, gp_ref):
    gate = jax.nn.sigmoid(_bdot(ctx_ref[...], stw_ref[...]) + stb_ref[...])
    x2 = x2_ref[...]
    y2 = gate * jnp.tanh(x2) + (1.0 - gate) * x2
    attn = _bdot(x1_ref[...], w1_ref[...]) + _bdot(y2, w2_ref[...]) + ob_ref[...]
    h = src_ref[...] + attn
    mu = jnp.mean(h, axis=-1, keepdims=True)
    var = jnp.mean((h - mu) ** 2, axis=-1, keepdims=True)
    h1 = (h - mu) / jnp.sqrt(var + 1e-5) * l1g_ref[...] + l1b_ref[...]
    h1_ref[...] = h1
    logits = _dot(h1, gw_ref[...]) + gb_ref[...]
    lm = jnp.max(logits, axis=-1, keepdims=True)
    ex = jnp.exp(logits - lm)
    gates = ex / jnp.sum(ex, axis=-1, keepdims=True)
    iot = lax.broadcasted_iota(jnp.int32, (L, E), 1)
    v1 = jnp.max(gates, axis=-1, keepdims=True)
    i1 = jnp.min(jnp.where(gates == v1, iot, E), axis=-1, keepdims=True)
    masked = jnp.where(iot == i1, -1.0, gates)
    v2 = jnp.max(masked, axis=-1, keepdims=True)
    i2 = jnp.min(jnp.where(masked == v2, iot, E), axis=-1, keepdims=True)
    oh1 = (iot == i1).astype(F32)
    oh2 = (iot == i2).astype(F32)
    tot = v1 + v2
    gp_ref[...] = jnp.concatenate([v1 / tot, v2 / tot], axis=1)
    m = oh1 + oh2
    msum = lax.dot_general(m, h1, (((0,), (0,)), ((), ())),
                           preferred_element_type=F32)
    cnt = lax.dot_general(m, jnp.ones((L, 1), F32), (((0,), (0,)), ((), ())),
                          preferred_element_type=F32)
    mean_ref[...] = msum / jnp.maximum(cnt, 1.0)
    cnt_row = jnp.sum(m, axis=0, keepdims=True)
    cnt_ref[...] = cnt_row
    CH = 256
    NCH = L // CH
    r_iot = lax.broadcasted_iota(jnp.int32, (CH, CH), 0)
    c_iot = lax.broadcasted_iota(jnp.int32, (CH, CH), 1)
    tri = (r_iot >= c_iot).astype(BF16)
    oh12 = jnp.concatenate([oh1, oh2], axis=1).astype(BF16)
    parts = []
    run = jnp.zeros((1, 2 * E), F32)
    for i in range(NCH):
        blk = oh12[i * CH:(i + 1) * CH, :]
        local = jnp.dot(tri, blk, preferred_element_type=F32)
        parts.append(local + run)
        run = run + local[CH - 1:CH, :]
    c12 = jnp.concatenate(parts, axis=0)
    c1, c2 = c12[:, :E], c12[:, E:]
    tot1 = jnp.max(c1, axis=0, keepdims=True)
    pc = jnp.ceil(cnt_row * (1.0 / BM)) * BM
    e_iot = lax.broadcasted_iota(jnp.int32, (E, E), 0)
    f_iot = lax.broadcasted_iota(jnp.int32, (E, E), 1)
    su = (e_iot < f_iot).astype(BF16)
    offs = jnp.dot(pc.astype(BF16), su, preferred_element_type=F32)
    slot0 = jnp.sum(oh1 * (offs + c1 - 1.0), axis=1, keepdims=True)
    slot1 = jnp.sum(oh2 * (offs + tot1 + c2 - 1.0), axis=1, keepdims=True)
    slot_ref[...] = jnp.concatenate([slot0, slot1], axis=1).astype(jnp.int32)


def _softplus(x):
    return jnp.maximum(x, 0.0) + jnp.log1p(jnp.exp(-jnp.abs(x)))


def _kd(mean_ref, ew1_ref, eb1_ref, eow_ref, eob_ref, epw_ref, epb_ref,
        om_ref, ph_ref):
    stats = _bdot(mean_ref[0], ew1_ref[0]) + eb1_ref[0]
    a = _bdot(stats, eow_ref[0]) + eob_ref[0]
    om_ref[0] = _softplus(a)
    ph_ref[0] = _bdot(stats, epw_ref[0]) + epb_ref[0]


def _gelu(x):
    return 0.5 * x * (1.0 + lax.erf(x * (1.0 / math.sqrt(2.0))))


def _kg(bexp_ref, xs_ref, om_ref, ph_ref, ew1_ref, eb1_ref, ew2_ref, eb2_ref,
        ys_ref):
    hh = _bdot(xs_ref[...], ew1_ref[0]) + eb1_ref[0]
    act = _gelu(om_ref[0] * hh + ph_ref[0])
    ys_ref[...] = _bdot(act, ew2_ref[0]) + eb2_ref[0]


def _kf(h1_ref, m_ref, gp_ref, l2g_ref, l2b_ref, out_ref):
    mf = m_ref[...]
    moe = mf[:, :D] * gp_ref[:, 0:1] + mf[:, D:] * gp_ref[:, 1:2]
    h = h1_ref[...] + moe
    mu = jnp.mean(h, axis=-1, keepdims=True)
    var = jnp.mean((h - mu) ** 2, axis=-1, keepdims=True)
    out_ref[...] = (h - mu) / jnp.sqrt(var + 1e-5) * l2g_ref[...] + l2b_ref[...]


def _sc_mesh():
    return plsc.VectorSubcoreMesh(core_axis_name="c", subcore_axis_name="s")


def _sc_disp_body(h1_hbm, tok_hbm, slot_hbm, xs_hbm, tok_v, slot_v, rows_v,
                  sem):
    wid = lax.axis_index("s") * SC_CORES + lax.axis_index("c")
    base = wid * PPW
    pltpu.sync_copy(tok_hbm.at[pl.ds(base, PPW)], tok_v)
    pltpu.sync_copy(slot_hbm.at[pl.ds(base, PPW)], slot_v)
    pltpu.async_copy(h1_hbm.at[tok_v], rows_v, sem).wait()
    pltpu.async_copy(rows_v, xs_hbm.at[slot_v], sem).wait()


def _sc_comb_body(ys_hbm, slot_hbm, m_hbm, slot_v, rows_v, sem):
    wid = lax.axis_index("s") * SC_CORES + lax.axis_index("c")
    base = wid * PPW
    pltpu.sync_copy(slot_hbm.at[pl.ds(base, PPW)], slot_v)
    pltpu.async_copy(ys_hbm.at[slot_v], rows_v, sem).wait()
    pltpu.sync_copy(rows_v, m_hbm.at[pl.ds(base, PPW)])


def _dispatch(h1, tok, slot):
    return pl.kernel(
        _sc_disp_body,
        mesh=_sc_mesh(),
        out_type=jax.ShapeDtypeStruct((NP, D), F32),
        scratch_types=[
            pltpu.VMEM((PPW,), jnp.int32),
            pltpu.VMEM((PPW,), jnp.int32),
            pltpu.VMEM((PPW, D), F32),
            pltpu.SemaphoreType.DMA,
        ],
    )(h1, tok, slot)


def _combine(ys, slot):
    return pl.kernel(
        _sc_comb_body,
        mesh=_sc_mesh(),
        out_type=jax.ShapeDtypeStruct((NPAIR, D), F32),
        scratch_types=[
            pltpu.VMEM((PPW,), jnp.int32),
            pltpu.VMEM((PPW, D), F32),
            pltpu.SemaphoreType.DMA,
        ],
    )(ys, slot)


def kernel(src, q_w, q_b, k_w, k_b, sal_w, sal_b, st_w, st_b, out_w, out_b,
           ln1_g, ln1_b, ln2_g, ln2_b, gate_w, gate_b,
           e_w1, e_b1, e_ow, e_ob, e_pw, e_pb, e_w2, e_b2):
    x = src[0]
    x1 = x[:, :DC]
    x2 = x[:, DC:]

    f32 = lambda s: jax.ShapeDtypeStruct(s, F32)

    Q, K, sal = pl.pallas_call(
        _ka,
        out_shape=[f32((L, DC)), f32((L, DC)), f32((L, H))],
    )(x1, q_w, q_b[None], k_w, k_b[None],
      sal_w[:, :, 0].T, sal_w[:, :, 1].T, sal_w[:, :, 2].T, sal_b[None])

    Qh = Q.reshape(L, H, DH).transpose(1, 0, 2)
    Kh = K.reshape(L, H, DH).transpose(1, 0, 2)
    salh = sal.T[:, None, :]

    attw, ctx = pl.pallas_call(
        _kb,
        grid=(H, L // BQ),
        in_specs=[
            pl.BlockSpec((1, BQ, DH), lambda h, q: (h, q, 0)),
            pl.BlockSpec((1, L, DH), lambda h, q: (h, 0, 0)),
            pl.BlockSpec((1, 1, L), lambda h, q: (h, 0, 0)),
        ],
        out_specs=[
            pl.BlockSpec((1, BQ, L), lambda h, q: (h, q, 0)),
            pl.BlockSpec((1, BQ, DH), lambda h, q: (h, q, 0)),
        ],
        out_shape=[f32((H, L, L)), f32((H, L, DH))],
    )(Qh, Kh, salh)

    ctxf = ctx.transpose(1, 0, 2).reshape(L, DC)

    h1, mean, cnt, slot2, gpair = pl.pallas_call(
        _kc,
        out_shape=[f32((L, D)), f32((E, D)), f32((1, E)),
                   jax.ShapeDtypeStruct((L, 2), jnp.int32), f32((L, 2))],
    )(ctxf, x1, x2, x, st_w, st_b[None], out_w[:DC], out_w[DC:], out_b[None],
      ln1_g[None], ln1_b[None], gate_w, gate_b[None])

    omega, phi = pl.pallas_call(
        _kd,
        grid=(E, 2),
        in_specs=[
            pl.BlockSpec((1, 1, D), lambda e, j: (e, 0, 0)),
            pl.BlockSpec((1, D, DFF), lambda e, j: (e, 0, 0)),
            pl.BlockSpec((1, 1, DFF), lambda e, j: (e, 0, 0)),
            pl.BlockSpec((1, DFF, DFF // 2), lambda e, j: (e, 0, j)),
            pl.BlockSpec((1, 1, DFF // 2), lambda e, j: (e, 0, j)),
            pl.BlockSpec((1, DFF, DFF // 2), lambda e, j: (e, 0, j)),
            pl.BlockSpec((1, 1, DFF // 2), lambda e, j: (e, 0, j)),
        ],
        out_specs=[
            pl.BlockSpec((1, 1, DFF // 2), lambda e, j: (e, 0, j)),
            pl.BlockSpec((1, 1, DFF // 2), lambda e, j: (e, 0, j)),
        ],
        out_shape=[f32((E, 1, DFF)), f32((E, 1, DFF))],
    )(mean[:, None, :], e_w1, e_b1[:, None, :], e_ow, e_ob[:, None, :],
      e_pw, e_pb[:, None, :])

    pc = jnp.ceil(cnt[0] * (1.0 / BM)) * BM
    ends = jnp.cumsum(pc)
    bexp = jnp.minimum(
        jnp.sum((lax.broadcasted_iota(F32, (NB, E), 0) * BM
                 >= ends[None, :]).astype(jnp.int32), axis=1),
        E - 1).astype(jnp.int32)

    slot = slot2.reshape(NPAIR)
    tok = lax.broadcasted_iota(jnp.int32, (NPAIR,), 0) // 2

    xs = _dispatch(h1, tok, slot)

    ys = pl.pallas_call(
        _kg,
        grid_spec=pltpu.PrefetchScalarGridSpec(
            num_scalar_prefetch=1,
            grid=(NB,),
            in_specs=[
                pl.BlockSpec((BM, D), lambda b, be: (b, 0)),
                pl.BlockSpec((1, 1, DFF), lambda b, be: (be[b], 0, 0)),
                pl.BlockSpec((1, 1, DFF), lambda b, be: (be[b], 0, 0)),
                pl.BlockSpec((1, D, DFF), lambda b, be: (be[b], 0, 0)),
                pl.BlockSpec((1, 1, DFF), lambda b, be: (be[b], 0, 0)),
                pl.BlockSpec((1, DFF, D), lambda b, be: (be[b], 0, 0)),
                pl.BlockSpec((1, 1, D), lambda b, be: (be[b], 0, 0)),
            ],
            out_specs=pl.BlockSpec((BM, D), lambda b, be: (b, 0)),
        ),
        out_shape=f32((NP, D)),
    )(bexp, xs, omega, phi, e_w1, e_b1[:, None, :], e_w2, e_b2[:, None, :])

    mm = _combine(ys, slot)

    out2d = pl.pallas_call(
        _kf,
        out_shape=f32((L, D)),
    )(h1, mm.reshape(L, 2 * D), gpair, ln2_g[None], ln2_b[None])

    return out2d[None], attw[None]


# speedup vs baseline: 1.0015x; 1.0015x over previous
import math

import jax
import jax.numpy as jnp
from jax import lax
from jax.experimental import pallas as pl
from jax.experimental.pallas import tpu as pltpu
from jax.experimental.pallas import tpu_sc as plsc

L, D = 2048, 768
DC = D // 2
H = 12
DH = DC // H
DFF = 1536
E = 8
BQ = 2048
BM = 256
NP = 2 * L + E * BM
NB = NP // BM
NPAIR = 2 * L
SC_CORES = 2
SC_SUBCORES = 16
NW = SC_CORES * SC_SUBCORES
PPW = NPAIR // NW
F32 = jnp.float32
BF16 = jnp.bfloat16


def _dot(a, b):
    return jnp.dot(a, b, preferred_element_type=F32)


def _bdot(a, b):
    return jnp.dot(a.astype(BF16), b.astype(BF16), preferred_element_type=F32)


def _ka(x1_ref, qw_ref, qb_ref, kw_ref, kb_ref, w0_ref, w1_ref, w2_ref,
        sb_ref, q_out, k_out, sal_out):
    x1 = x1_ref[...]
    q_out[...] = (_bdot(x1, qw_ref[...]) + qb_ref[...]) * (1.0 / math.sqrt(DH))
    k_out[...] = _bdot(x1, kw_ref[...]) + kb_ref[...]
    a0 = _bdot(x1, w0_ref[...])
    a1 = _bdot(x1, w1_ref[...])
    a2 = _bdot(x1, w2_ref[...])
    z = jnp.zeros((1, H), F32)
    sal_out[...] = (jnp.concatenate([z, a0[:-1]], axis=0) + a1
                    + jnp.concatenate([a2[1:], z], axis=0) + sb_ref[...])


def _kb(q_ref, k_ref, sal_ref, attw_ref, ctx_ref):
    q = q_ref[0]
    k = k_ref[0]
    s = _bdot(q, k.T) + sal_ref[0]
    m = jnp.max(s, axis=-1, keepdims=True)
    p = jnp.exp(s - m)
    attw = p * (1.0 / jnp.sum(p, axis=-1, keepdims=True))
    attw_ref[0] = attw
    ctx_ref[0] = _bdot(attw, k)


def _kc(ctx_ref, x1_ref, x2_ref, src_ref, stw_ref, stb_ref, w1_ref, w2_ref,
        ob_ref, l1g_ref, l1b_ref, gw_ref, gb_ref,
        h1_ref, mean_ref, cnt_ref, slot_ref, gp_ref):
    gate = jax.nn.sigmoid(_bdot(ctx_ref[...], stw_ref[...]) + stb_ref[...])
    x2 = x2_ref[...]
    y2 = gate * jnp.tanh(x2) + (1.0 - gate) * x2
    attn = _bdot(x1_ref[...], w1_ref[...]) + _bdot(y2, w2_ref[...]) + ob_ref[...]
    h = src_ref[...] + attn
    mu = jnp.mean(h, axis=-1, keepdims=True)
    var = jnp.mean((h - mu) ** 2, axis=-1, keepdims=True)
    h1 = (h - mu) / jnp.sqrt(var + 1e-5) * l1g_ref[...] + l1b_ref[...]
    h1_ref[...] = h1
    logits = _dot(h1, gw_ref[...]) + gb_ref[...]
    lm = jnp.max(logits, axis=-1, keepdims=True)
    ex = jnp.exp(logits - lm)
    gates = ex / jnp.sum(ex, axis=-1, keepdims=True)
    iot = lax.broadcasted_iota(jnp.int32, (L, E), 1)
    v1 = jnp.max(gates, axis=-1, keepdims=True)
    i1 = jnp.min(jnp.where(gates == v1, iot, E), axis=-1, keepdims=True)
    masked = jnp.where(iot == i1, -1.0, gates)
    v2 = jnp.max(masked, axis=-1, keepdims=True)
    i2 = jnp.min(jnp.where(masked == v2, iot, E), axis=-1, keepdims=True)
    oh1 = (iot == i1).astype(F32)
    oh2 = (iot == i2).astype(F32)
    tot = v1 + v2
    gp_ref[...] = jnp.concatenate([v1 / tot, v2 / tot], axis=1)
    m = oh1 + oh2
    msum = lax.dot_general(m, h1, (((0,), (0,)), ((), ())),
                           preferred_element_type=F32)
    cnt = lax.dot_general(m, jnp.ones((L, 1), F32), (((0,), (0,)), ((), ())),
                          preferred_element_type=F32)
    mean_ref[...] = msum / jnp.maximum(cnt, 1.0)
    cnt_row = jnp.sum(m, axis=0, keepdims=True)
    cnt_ref[...] = cnt_row
    CH = 256
    NCH = L // CH
    r_iot = lax.broadcasted_iota(jnp.int32, (CH, CH), 0)
    c_iot = lax.broadcasted_iota(jnp.int32, (CH, CH), 1)
    tri = (r_iot >= c_iot).astype(BF16)
    oh12 = jnp.concatenate([oh1, oh2], axis=1).astype(BF16)
    parts = []
    run = jnp.zeros((1, 2 * E), F32)
    for i in range(NCH):
        blk = oh12[i * CH:(i + 1) * CH, :]
        local = jnp.dot(tri, blk, preferred_element_type=F32)
        parts.append(local + run)
        run = run + local[CH - 1:CH, :]
    c12 = jnp.concatenate(parts, axis=0)
    c1, c2 = c12[:, :E], c12[:, E:]
    tot1 = jnp.max(c1, axis=0, keepdims=True)
    pc = jnp.ceil(cnt_row * (1.0 / BM)) * BM
    e_iot = lax.broadcasted_iota(jnp.int32, (E, E), 0)
    f_iot = lax.broadcasted_iota(jnp.int32, (E, E), 1)
    su = (e_iot < f_iot).astype(BF16)
    offs = jnp.dot(pc.astype(BF16), su, preferred_element_type=F32)
    slot0 = jnp.sum(oh1 * (offs + c1 - 1.0), axis=1, keepdims=True)
    slot1 = jnp.sum(oh2 * (offs + tot1 + c2 - 1.0), axis=1, keepdims=True)
    slot_ref[...] = jnp.concatenate([slot0, slot1], axis=1).astype(jnp.int32)


def _softplus(x):
    return jnp.maximum(x, 0.0) + jnp.log1p(jnp.exp(-jnp.abs(x)))


def _kd(mean_ref, ew1_ref, eb1_ref, eow_ref, eob_ref, epw_ref, epb_ref,
        om_ref, ph_ref):
    stats = _bdot(mean_ref[0], ew1_ref[0]) + eb1_ref[0]
    a = _bdot(stats, eow_ref[0]) + eob_ref[0]
    om_ref[0] = _softplus(a)
    ph_ref[0] = _bdot(stats, epw_ref[0]) + epb_ref[0]


def _gelu(x):
    return 0.5 * x * (1.0 + lax.erf(x * (1.0 / math.sqrt(2.0))))


def _kg(bexp_ref, xs_ref, om_ref, ph_ref, ew1_ref, eb1_ref, ew2_ref, eb2_ref,
        ys_ref):
    hh = _bdot(xs_ref[...], ew1_ref[0]) + eb1_ref[0]
    act = _gelu(om_ref[0] * hh + ph_ref[0])
    ys_ref[...] = _bdot(act, ew2_ref[0]) + eb2_ref[0]


def _kf(h1_ref, m_ref, gp_ref, l2g_ref, l2b_ref, out_ref):
    mf = m_ref[...]
    moe = mf[:, :D] * gp_ref[:, 0:1] + mf[:, D:] * gp_ref[:, 1:2]
    h = h1_ref[...] + moe
    mu = jnp.mean(h, axis=-1, keepdims=True)
    var = jnp.mean((h - mu) ** 2, axis=-1, keepdims=True)
    out_ref[...] = (h - mu) / jnp.sqrt(var + 1e-5) * l2g_ref[...] + l2b_ref[...]


def _sc_mesh():
    return plsc.VectorSubcoreMesh(core_axis_name="c", subcore_axis_name="s")


def _sc_disp_body(h1_hbm, tok_hbm, slot_hbm, xs_hbm, tok_v, slot_v, rows_v,
                  sem):
    wid = lax.axis_index("s") * SC_CORES + lax.axis_index("c")
    base = wid * PPW
    pltpu.sync_copy(tok_hbm.at[pl.ds(base, PPW)], tok_v)
    pltpu.sync_copy(slot_hbm.at[pl.ds(base, PPW)], slot_v)
    pltpu.async_copy(h1_hbm.at[tok_v], rows_v, sem).wait()
    pltpu.async_copy(rows_v, xs_hbm.at[slot_v], sem).wait()


def _sc_comb_body(ys_hbm, slot_hbm, m_hbm, slot_v, rows_v, sem):
    wid = lax.axis_index("s") * SC_CORES + lax.axis_index("c")
    base = wid * PPW
    pltpu.sync_copy(slot_hbm.at[pl.ds(base, PPW)], slot_v)
    pltpu.async_copy(ys_hbm.at[slot_v], rows_v, sem).wait()
    pltpu.sync_copy(rows_v, m_hbm.at[pl.ds(base, PPW)])


def _dispatch(h1, tok, slot):
    return pl.kernel(
        _sc_disp_body,
        mesh=_sc_mesh(),
        out_type=jax.ShapeDtypeStruct((NP, D), F32),
        scratch_types=[
            pltpu.VMEM((PPW,), jnp.int32),
            pltpu.VMEM((PPW,), jnp.int32),
            pltpu.VMEM((PPW, D), F32),
            pltpu.SemaphoreType.DMA,
        ],
    )(h1, tok, slot)


def _combine(ys, slot):
    return pl.kernel(
        _sc_comb_body,
        mesh=_sc_mesh(),
        out_type=jax.ShapeDtypeStruct((NPAIR, D), F32),
        scratch_types=[
            pltpu.VMEM((PPW,), jnp.int32),
            pltpu.VMEM((PPW, D), F32),
            pltpu.SemaphoreType.DMA,
        ],
    )(ys, slot)


def kernel(src, q_w, q_b, k_w, k_b, sal_w, sal_b, st_w, st_b, out_w, out_b,
           ln1_g, ln1_b, ln2_g, ln2_b, gate_w, gate_b,
           e_w1, e_b1, e_ow, e_ob, e_pw, e_pb, e_w2, e_b2):
    x = src[0]
    x1 = x[:, :DC]
    x2 = x[:, DC:]

    f32 = lambda s: jax.ShapeDtypeStruct(s, F32)

    Q, K, sal = pl.pallas_call(
        _ka,
        out_shape=[f32((L, DC)), f32((L, DC)), f32((L, H))],
    )(x1, q_w, q_b[None], k_w, k_b[None],
      sal_w[:, :, 0].T, sal_w[:, :, 1].T, sal_w[:, :, 2].T, sal_b[None])

    Qh = Q.reshape(L, H, DH).transpose(1, 0, 2)
    Kh = K.reshape(L, H, DH).transpose(1, 0, 2)
    salh = sal.T[:, None, :]

    attw, ctx = pl.pallas_call(
        _kb,
        grid=(H, L // BQ),
        in_specs=[
            pl.BlockSpec((1, BQ, DH), lambda h, q: (h, q, 0)),
            pl.BlockSpec((1, L, DH), lambda h, q: (h, 0, 0)),
            pl.BlockSpec((1, 1, L), lambda h, q: (h, 0, 0)),
        ],
        out_specs=[
            pl.BlockSpec((1, BQ, L), lambda h, q: (h, q, 0)),
            pl.BlockSpec((1, BQ, DH), lambda h, q: (h, q, 0)),
        ],
        out_shape=[f32((H, L, L)), f32((H, L, DH))],
    )(Qh, Kh, salh)

    ctxf = ctx.transpose(1, 0, 2).reshape(L, DC)

    h1, mean, cnt, slot2, gpair = pl.pallas_call(
        _kc,
        out_shape=[f32((L, D)), f32((E, D)), f32((1, E)),
                   jax.ShapeDtypeStruct((L, 2), jnp.int32), f32((L, 2))],
    )(ctxf, x1, x2, x, st_w, st_b[None], out_w[:DC], out_w[DC:], out_b[None],
      ln1_g[None], ln1_b[None], gate_w, gate_b[None])

    omega, phi = pl.pallas_call(
        _kd,
        grid=(E, 2),
        in_specs=[
            pl.BlockSpec((1, 1, D), lambda e, j: (e, 0, 0)),
            pl.BlockSpec((1, D, DFF), lambda e, j: (e, 0, 0)),
            pl.BlockSpec((1, 1, DFF), lambda e, j: (e, 0, 0)),
            pl.BlockSpec((1, DFF, DFF // 2), lambda e, j: (e, 0, j)),
            pl.BlockSpec((1, 1, DFF // 2), lambda e, j: (e, 0, j)),
            pl.BlockSpec((1, DFF, DFF // 2), lambda e, j: (e, 0, j)),
            pl.BlockSpec((1, 1, DFF // 2), lambda e, j: (e, 0, j)),
        ],
        out_specs=[
            pl.BlockSpec((1, 1, DFF // 2), lambda e, j: (e, 0, j)),
            pl.BlockSpec((1, 1, DFF // 2), lambda e, j: (e, 0, j)),
        ],
        out_shape=[f32((E, 1, DFF)), f32((E, 1, DFF))],
    )(mean[:, None, :], e_w1, e_b1[:, None, :], e_ow, e_ob[:, None, :],
      e_pw, e_pb[:, None, :])

    pc = jnp.ceil(cnt[0] * (1.0 / BM)) * BM
    ends = jnp.cumsum(pc)
    bexp = jnp.minimum(
        jnp.sum((lax.broadcasted_iota(F32, (NB, E), 0) * BM
                 >= ends[None, :]).astype(jnp.int32), axis=1),
        E - 1).astype(jnp.int32)

    slot = slot2.reshape(NPAIR)
    tok = lax.broadcasted_iota(jnp.int32, (NPAIR,), 0) // 2

    xs = _dispatch(h1, tok, slot)

    ys = pl.pallas_call(
        _kg,
        grid_spec=pltpu.PrefetchScalarGridSpec(
            num_scalar_prefetch=1,
            grid=(NB,),
            in_specs=[
                pl.BlockSpec((BM, D), lambda b, be: (b, 0)),
                pl.BlockSpec((1, 1, DFF), lambda b, be: (be[b], 0, 0)),
                pl.BlockSpec((1, 1, DFF), lambda b, be: (be[b], 0, 0)),
                pl.BlockSpec((1, D, DFF), lambda b, be: (be[b], 0, 0)),
                pl.BlockSpec((1, 1, DFF), lambda b, be: (be[b], 0, 0)),
                pl.BlockSpec((1, DFF, D), lambda b, be: (be[b], 0, 0)),
                pl.BlockSpec((1, 1, D), lambda b, be: (be[b], 0, 0)),
            ],
            out_specs=pl.BlockSpec((BM, D), lambda b, be: (b, 0)),
        ),
        out_shape=f32((NP, D)),
    )(bexp, xs, omega, phi, e_w1, e_b1[:, None, :], e_w2, e_b2[:, None, :])

    mm = _combine(ys, slot)

    out2d = pl.pallas_call(
        _kf,
        out_shape=f32((L, D)),
    )(h1, mm.reshape(L, 2 * D), gpair, ln2_g[None], ln2_b[None])

    return out2d[None], attw[None]
